# baseline (speedup 1.0000x reference)
import functools

import jax
import jax.numpy as jnp
from jax import lax
from jax.experimental import pallas as pl
from jax.experimental.pallas import tpu as pltpu
from jax.experimental.pallas import tpu_sc as plsc


def kernel(x):
    B, F, K = x.shape
    P = F * (F - 1) // 2
    XW = F * K

    info = plsc.get_sparse_core_info()
    NC, NS, L = info.num_cores, info.num_subcores, info.num_lanes
    NW = NC * NS
    BLK = L
    SAMP_PER_W = B // NW
    NBLK = SAMP_PER_W // BLK

    xf = x.reshape(-1)
    mesh = plsc.VectorSubcoreMesh(core_axis_name="c", subcore_axis_name="s")

    @functools.partial(
        pl.kernel,
        mesh=mesh,
        out_type=jax.ShapeDtypeStruct((B * P,), jnp.float32),
        scratch_types=[
            pltpu.VMEM((BLK * XW,), jnp.float32),
            pltpu.VMEM((BLK * P,), jnp.float32),
        ],
        compiler_params=pltpu.CompilerParams(needs_layout_passes=False),
    )
    def run(x_hbm, out_hbm, x_v, o_v):
        wid = lax.axis_index("s") * NC + lax.axis_index("c")
        lane = lax.broadcasted_iota(jnp.int32, (L,), 0)
        ib = lane * XW
        ob = lane * P

        def block_body(t, carry):
            base = (wid * NBLK + t) * BLK
            pltpu.sync_copy(x_hbm.at[pl.ds(base * XW, BLK * XW)], x_v)
            for i in range(F - 1):
                xi = [plsc.load_gather(x_v, [ib + (i * K + k)]) for k in range(K)]
                pconst = i * F - (i * (i + 1)) // 2 - i - 1

                def jbody(j, c, xi=xi):
                    off = j * K
                    acc = None
                    for k in range(K):
                        xj = plsc.load_gather(x_v, [ib + (off + k)])
                        prod = xi[k] * xj
                        acc = prod if acc is None else acc + prod
                    plsc.store_scatter(o_v, [ob + (pconst + j)], acc)
                    return c

                lax.fori_loop(i + 1, F, jbody, 0)
            pltpu.sync_copy(o_v, out_hbm.at[pl.ds(base * P, BLK * P)])
            return carry

        lax.fori_loop(0, NBLK, block_body, 0)

    return run(xf).reshape(B, P)
